# baseline (speedup 1.0000x reference)
import jax
import jax.numpy as jnp
from jax.experimental import pallas as pl
from jax.experimental.pallas import tpu as pltpu
from jax.experimental.pallas import tpu_sc as plsc

_WIN = 256


def _tab_kernel(ce_ref, pos_ref, gam_ref, bet_ref, tab_ref):
    ce = ce_ref[...]
    pos = pos_ref[...]
    x = pos[:, None, :] + ce[None, :, :]
    mu = jnp.mean(x, axis=-1, keepdims=True)
    xc = x - mu
    var = jnp.mean(xc * xc, axis=-1, keepdims=True)
    xn = xc * jax.lax.rsqrt(var + 1e-5)
    tab_ref[...] = xn * gam_ref[0][None, None, :] + bet_ref[0][None, None, :]


def _sc_gather_call(tabp, idx_flat, n_rows, d2):
    n_win = n_rows // _WIN
    mesh = plsc.VectorSubcoreMesh(core_axis_name="c", subcore_axis_name="s")

    @pl.kernel(
        out_type=jax.ShapeDtypeStruct((n_rows, d2), jnp.float32), mesh=mesh
    )
    def k(tab_hbm, i_hbm, o_hbm):
        def body(i_vmem, o_vmem):
            pltpu.sync_copy(tab_hbm.at[i_vmem.at[0]], o_vmem)

        pltpu.emit_pipeline(
            body,
            grid=(n_win,),
            in_specs=[pl.BlockSpec((1, _WIN), lambda i: (0, i))],
            out_specs=[pl.BlockSpec((_WIN, d2), lambda i: (i, 0))],
            core_axis_name=("c", "s"),
            dimension_semantics=(pltpu.PARALLEL,),
        )(i_hbm, o_hbm)

    return k(tabp, idx_flat)


def kernel(grid, color_embed, pos_embed, gamma, beta):
    B, H, W = grid.shape
    P = H * W
    C, D = color_embed.shape
    Q = P // 2

    posf = pos_embed[0, :H, :W, :].reshape(P, D)

    tab = pl.pallas_call(
        _tab_kernel,
        out_shape=jax.ShapeDtypeStruct((P, C, D), jnp.float32),
    )(color_embed, posf, gamma.reshape(1, D), beta.reshape(1, D))

    tabe = tab[0::2]
    tabo = tab[1::2]
    tabp = jnp.concatenate(
        [
            jnp.broadcast_to(tabe[:, :, None, :], (Q, C, C, D)),
            jnp.broadcast_to(tabo[:, None, :, :], (Q, C, C, D)),
        ],
        axis=-1,
    ).reshape(Q * C * C, 2 * D)

    g2 = grid.reshape(B, P)
    qbase = (C * C) * jnp.arange(Q, dtype=jnp.int32)[None, :]
    idx = qbase + C * g2[:, 0::2] + g2[:, 1::2]
    idx_flat = idx.reshape(1, B * Q)

    out = _sc_gather_call(tabp, idx_flat, B * Q, 2 * D)
    return out.reshape(B, P, D)


# speedup vs baseline: 1.0242x; 1.0242x over previous
import functools

import jax
import jax.numpy as jnp
from jax.experimental import pallas as pl
from jax.experimental.pallas import tpu as pltpu
from jax.experimental.pallas import tpu_sc as plsc

_BB = 32
_WIN = 256
_B_TC = 2304


def _tab_kernel(ce_ref, pos_ref, gam_ref, bet_ref, tab_ref):
    ce = ce_ref[...]
    pos = pos_ref[...]
    x = pos[:, None, :] + ce[None, :, :]
    mu = jnp.mean(x, axis=-1, keepdims=True)
    xc = x - mu
    var = jnp.mean(xc * xc, axis=-1, keepdims=True)
    xn = xc * jax.lax.rsqrt(var + 1e-5)
    tab_ref[...] = xn * gam_ref[0][None, None, :] + bet_ref[0][None, None, :]


def _select_kernel(g_ref, tab_ref, out_ref, *, num_colors):
    g = g_ref[...]
    gexp = jnp.repeat(g, 64, axis=1)
    tab = tab_ref[...]
    acc = jnp.broadcast_to(tab[0:1, :], gexp.shape)
    for c in range(1, num_colors):
        acc = jnp.where(gexp == c, jnp.broadcast_to(tab[c:c + 1, :], gexp.shape), acc)
    out_ref[...] = acc


def _sc_gather_call(tabp, idx_flat, n_rows, d2):
    n_win = n_rows // _WIN
    mesh = plsc.VectorSubcoreMesh(core_axis_name="c", subcore_axis_name="s")

    @pl.kernel(
        out_type=jax.ShapeDtypeStruct((n_rows, d2), jnp.float32), mesh=mesh
    )
    def k(tab_hbm, i_hbm, o_hbm):
        def body(i_vmem, o_vmem):
            pltpu.sync_copy(tab_hbm.at[i_vmem.at[0]], o_vmem)

        pltpu.emit_pipeline(
            body,
            grid=(n_win,),
            in_specs=[pl.BlockSpec((1, _WIN), lambda i: (0, i))],
            out_specs=[pl.BlockSpec((_WIN, d2), lambda i: (i, 0))],
            core_axis_name=("c", "s"),
            dimension_semantics=(pltpu.PARALLEL,),
        )(i_hbm, o_hbm)

    return k(tabp, idx_flat)


def kernel(grid, color_embed, pos_embed, gamma, beta):
    B, H, W = grid.shape
    P = H * W
    C, D = color_embed.shape
    Q = P // 2
    b_tc = min(_B_TC, B)
    b_sc = B - b_tc

    posf = pos_embed[0, :H, :W, :].reshape(P, D)

    tab = pl.pallas_call(
        _tab_kernel,
        out_shape=jax.ShapeDtypeStruct((P, C, D), jnp.float32),
    )(color_embed, posf, gamma.reshape(1, D), beta.reshape(1, D))

    g2 = grid.reshape(B, P)
    outs = []

    if b_tc:
        tabf = tab.transpose(1, 0, 2).reshape(C, P * D)
        out_tc = pl.pallas_call(
            functools.partial(_select_kernel, num_colors=C),
            grid=(b_tc // _BB,),
            in_specs=[
                pl.BlockSpec((_BB, P), lambda i: (i, 0)),
                pl.BlockSpec((C, P * D), lambda i: (0, 0)),
            ],
            out_specs=pl.BlockSpec((_BB, P * D), lambda i: (i, 0)),
            out_shape=jax.ShapeDtypeStruct((b_tc, P * D), jnp.float32),
        )(g2[:b_tc], tabf)
        outs.append(out_tc.reshape(b_tc, P, D))

    if b_sc:
        tabe = tab[0::2]
        tabo = tab[1::2]
        tabp = jnp.concatenate(
            [
                jnp.broadcast_to(tabe[:, :, None, :], (Q, C, C, D)),
                jnp.broadcast_to(tabo[:, None, :, :], (Q, C, C, D)),
            ],
            axis=-1,
        ).reshape(Q * C * C, 2 * D)

        gt = g2[b_tc:]
        qbase = (C * C) * jnp.arange(Q, dtype=jnp.int32)[None, :]
        idx = qbase + C * gt[:, 0::2] + gt[:, 1::2]
        idx_flat = idx.reshape(1, b_sc * Q)

        out_sc = _sc_gather_call(tabp, idx_flat, b_sc * Q, 2 * D)
        outs.append(out_sc.reshape(b_sc, P, D))

    if len(outs) == 1:
        return outs[0]
    return jnp.concatenate(outs, axis=0)


# speedup vs baseline: 1.6919x; 1.6519x over previous
import functools

import jax
import jax.numpy as jnp
from jax.experimental import pallas as pl
from jax.experimental.pallas import tpu as pltpu
from jax.experimental.pallas import tpu_sc as plsc

_BB = 32
_WIN = 256
_B_TC = 0


def _tab_kernel(ce_ref, pos_ref, gam_ref, bet_ref, tab_ref):
    ce = ce_ref[...]
    pos = pos_ref[...]
    x = pos[:, None, :] + ce[None, :, :]
    mu = jnp.mean(x, axis=-1, keepdims=True)
    xc = x - mu
    var = jnp.mean(xc * xc, axis=-1, keepdims=True)
    xn = xc * jax.lax.rsqrt(var + 1e-5)
    tab_ref[...] = xn * gam_ref[0][None, None, :] + bet_ref[0][None, None, :]


def _select_kernel(g_ref, tab_ref, out_ref, *, num_colors):
    g = g_ref[...]
    gexp = jnp.repeat(g, 64, axis=1)
    tab = tab_ref[...]
    acc = jnp.broadcast_to(tab[0:1, :], gexp.shape)
    for c in range(1, num_colors):
        acc = jnp.where(gexp == c, jnp.broadcast_to(tab[c:c + 1, :], gexp.shape), acc)
    out_ref[...] = acc


def _sc_gather_call(tabp, idx_flat, n_rows, d2):
    mesh = plsc.VectorSubcoreMesh(core_axis_name="c", subcore_axis_name="s")

    n_b, n_q = n_rows
    @pl.kernel(
        out_type=jax.ShapeDtypeStruct((n_b, n_q, d2), jnp.float32), mesh=mesh
    )
    def k(tab_hbm, i_hbm, o_hbm):
        def body(i_vmem, o_vmem):
            pltpu.sync_copy(tab_hbm.at[i_vmem.at[0]], o_vmem.at[0])

        pltpu.emit_pipeline(
            body,
            grid=(n_b * n_q // _WIN,),
            in_specs=[pl.BlockSpec((1, _WIN), lambda i: (0, i))],
            out_specs=[pl.BlockSpec((1, _WIN, d2),
                                    lambda i: (i // (n_q // _WIN), i % (n_q // _WIN), 0))],
            core_axis_name=("c", "s"),
            dimension_semantics=(pltpu.PARALLEL,),
        )(i_hbm, o_hbm)

    return k(tabp, idx_flat)


def kernel(grid, color_embed, pos_embed, gamma, beta):
    B, H, W = grid.shape
    P = H * W
    C, D = color_embed.shape
    Q = P // 2
    b_tc = min(_B_TC, B)
    b_sc = B - b_tc

    posf = pos_embed[0, :H, :W, :].reshape(P, D)

    tab = pl.pallas_call(
        _tab_kernel,
        out_shape=jax.ShapeDtypeStruct((P, C, D), jnp.float32),
    )(color_embed, posf, gamma.reshape(1, D), beta.reshape(1, D))

    g2 = grid.reshape(B, P)
    outs = []

    if b_tc:
        tabf = tab.transpose(1, 0, 2).reshape(C, P * D)
        out_tc = pl.pallas_call(
            functools.partial(_select_kernel, num_colors=C),
            grid=(b_tc // _BB,),
            in_specs=[
                pl.BlockSpec((_BB, P), lambda i: (i, 0)),
                pl.BlockSpec((C, P * D), lambda i: (0, 0)),
            ],
            out_specs=pl.BlockSpec((_BB, P * D), lambda i: (i, 0)),
            out_shape=jax.ShapeDtypeStruct((b_tc, P * D), jnp.float32),
        )(g2[:b_tc], tabf)
        outs.append(out_tc.reshape(b_tc, P, D))

    if b_sc:
        tabe = tab[0::2]
        tabo = tab[1::2]
        tabp = jnp.concatenate(
            [
                jnp.broadcast_to(tabe[:, :, None, :], (Q, C, C, D)),
                jnp.broadcast_to(tabo[:, None, :, :], (Q, C, C, D)),
            ],
            axis=-1,
        ).reshape(Q * C * C, 2 * D)

        gt = g2[b_tc:]
        qbase = (C * C) * jnp.arange(Q, dtype=jnp.int32)[None, :]
        idx = qbase + C * gt[:, 0::2] + gt[:, 1::2]
        idx_flat = idx.reshape(1, b_sc * Q)

        out_sc = _sc_gather_call(tabp, idx_flat, (b_sc, Q), 2 * D)
        outs.append(out_sc.reshape(b_sc, P, D))

    if len(outs) == 1:
        return outs[0]
    return jnp.concatenate(outs, axis=0)
